# baseline (speedup 1.0000x reference)
import functools

import jax
import jax.numpy as jnp
from jax import lax
from jax.experimental import pallas as pl
from jax.experimental.pallas import tpu as pltpu
from jax.experimental.pallas import tpu_sc as plsc

NC = 2
NS = 16
L = 16
NW = NC * NS

B = 16384
D = 32
PW = B // NW
GROUPS = PW // L

_mesh = plsc.VectorSubcoreMesh(core_axis_name="c", subcore_axis_name="s")


@functools.partial(
    pl.kernel,
    out_type=jax.ShapeDtypeStruct((B,), jnp.float32),
    mesh=_mesh,
    scratch_types=[
        pltpu.VMEM((PW,), jnp.int32),
        pltpu.VMEM((PW,), jnp.int32),
        pltpu.VMEM((PW, D), jnp.float32),
        pltpu.VMEM((PW, D), jnp.float32),
        pltpu.VMEM((PW,), jnp.float32),
        pltpu.VMEM((PW,), jnp.float32),
        pltpu.VMEM((L,), jnp.float32),
        pltpu.VMEM((PW,), jnp.float32),
        pltpu.SemaphoreType.DMA,
        pltpu.SemaphoreType.DMA,
        pltpu.SemaphoreType.DMA,
        pltpu.SemaphoreType.DMA,
    ],
    compiler_params=pltpu.CompilerParams(
        use_tc_tiling_on_sc=False, needs_layout_passes=False
    ),
)
def _sc_predict(uidx_hbm, iidx_hbm, uembd_hbm, iembd_hbm, ubias_hbm,
                ibias_hbm, oab_hbm, out_hbm,
                uidx_v, iidx_v, urows_v, irows_v, ubias_v, ibias_v,
                oab_v, out_v, sem_u, sem_i, sem_bu, sem_bi):
    wid = lax.axis_index("s") * NC + lax.axis_index("c")
    base = wid * PW

    pltpu.sync_copy(uidx_hbm.at[pl.ds(base, PW)], uidx_v)
    pltpu.sync_copy(iidx_hbm.at[pl.ds(base, PW)], iidx_v)
    pltpu.sync_copy(oab_hbm, oab_v)

    cu = pltpu.async_copy(uembd_hbm.at[uidx_v], urows_v, sem_u)
    ci = pltpu.async_copy(iembd_hbm.at[iidx_v], irows_v, sem_i)
    cbu = pltpu.async_copy(ubias_hbm.at[uidx_v], ubias_v, sem_bu)
    cbi = pltpu.async_copy(ibias_hbm.at[iidx_v], ibias_v, sem_bi)
    cu.wait()
    ci.wait()
    cbu.wait()
    cbi.wait()

    lanes = lax.iota(jnp.int32, L)
    oab = oab_v[...]

    def group(g, carry):
        rows = g * L + lanes
        acc = ubias_v[pl.ds(g * L, L)] + ibias_v[pl.ds(g * L, L)] + oab
        for d in range(D):
            col = jnp.full((L,), d, jnp.int32)
            uc = plsc.load_gather(urows_v, [rows, col])
            ic = plsc.load_gather(irows_v, [rows, col])
            acc = acc + uc * ic
        out_v[pl.ds(g * L, L)] = acc
        return carry

    lax.fori_loop(0, GROUPS, group, 0)

    pltpu.sync_copy(out_v, out_hbm.at[pl.ds(base, PW)])


def kernel(userIdx, itemIdx, uEmbd, iEmbd, uBias, iBias, overAllBias):
    uidx = userIdx.astype(jnp.int32)
    iidx = itemIdx.astype(jnp.int32)
    ubias = uBias.reshape(-1)
    ibias = iBias.reshape(-1)
    oab = jnp.broadcast_to(overAllBias.astype(jnp.float32), (L,))
    return _sc_predict(uidx, iidx, uEmbd, iEmbd, ubias, ibias, oab)


# speedup vs baseline: 2.7453x; 2.7453x over previous
import functools

import jax
import jax.numpy as jnp
from jax import lax
from jax.experimental import pallas as pl
from jax.experimental.pallas import tpu as pltpu
from jax.experimental.pallas import tpu_sc as plsc

NC = 2
NS = 16
L = 16
NW = NC * NS

B = 16384
D = 32
SUB = 8
PLANES = D // SUB
PW = B // NW
GROUPS = PW // L
CH = 64
NCH = PW // CH
KB = 4

_mesh = plsc.VectorSubcoreMesh(core_axis_name="c", subcore_axis_name="s")


@functools.partial(
    pl.kernel,
    out_type=jax.ShapeDtypeStruct((B,), jnp.float32),
    mesh=_mesh,
    scratch_types=[
        pltpu.VMEM((PW,), jnp.int32),
        pltpu.VMEM((PW,), jnp.int32),
        pltpu.VMEM((PLANES, SUB, CH * L), jnp.float32),
        pltpu.VMEM((PLANES, SUB, CH * L), jnp.float32),
        pltpu.VMEM((PW * L,), jnp.float32),
        pltpu.VMEM((PW * L,), jnp.float32),
        pltpu.VMEM((L,), jnp.float32),
        pltpu.VMEM((PW,), jnp.float32),
        pltpu.SemaphoreType.DMA,
        pltpu.SemaphoreType.DMA,
    ],
    compiler_params=pltpu.CompilerParams(needs_layout_passes=False),
)
def _sc_predict(uidx_hbm, iidx_hbm, uembd_hbm, iembd_hbm, ubias_hbm,
                ibias_hbm, oab_hbm, out_hbm,
                uidx_v, iidx_v, useg_v, iseg_v, ubias_v, ibias_v,
                oab_v, out_v, sem_rows, sem_bias):
    wid = lax.axis_index("s") * NC + lax.axis_index("c")
    base = wid * PW

    pltpu.sync_copy(uidx_hbm.at[pl.ds(base, PW)], uidx_v)
    pltpu.sync_copy(iidx_hbm.at[pl.ds(base, PW)], iidx_v)
    pltpu.sync_copy(oab_hbm, oab_v)

    oab = oab_v[...]
    lanes = lax.iota(jnp.int32, L)

    def bias_batch(bi, carry):
        j0 = bi * L
        uvec = uidx_v[pl.ds(j0, L)]
        ivec = iidx_v[pl.ds(j0, L)]
        copies = []
        for t in range(L):
            j = j0 + t
            vu16 = pl.multiple_of((uvec[t] >> 4) << 4, L)
            vi16 = pl.multiple_of((ivec[t] >> 4) << 4, L)
            dj = pl.multiple_of(j * L, L)
            copies.append(pltpu.async_copy(
                ubias_hbm.at[pl.ds(vu16, L)],
                ubias_v.at[pl.ds(dj, L)], sem_bias))
            copies.append(pltpu.async_copy(
                ibias_hbm.at[pl.ds(vi16, L)],
                ibias_v.at[pl.ds(dj, L)], sem_bias))
            if (t + 1) % KB == 0:
                for c in copies:
                    c.wait()
                copies = []
        return carry

    lax.fori_loop(0, GROUPS, bias_batch, 0)

    def chunk(ci, carry):
        c0 = ci * CH

        def gather_batch(bi, carry2):
            j0 = c0 + bi * L
            uvec = uidx_v[pl.ds(j0, L)]
            ivec = iidx_v[pl.ds(j0, L)]
            copies = []
            for t in range(L):
                jj = bi * L + t
                vu16 = pl.multiple_of((uvec[t] >> 4) << 4, L)
                vi16 = pl.multiple_of((ivec[t] >> 4) << 4, L)
                dj = pl.multiple_of(jj * L, L)
                copies.append(pltpu.async_copy(
                    uembd_hbm.at[:, :, pl.ds(vu16, L)],
                    useg_v.at[:, :, pl.ds(dj, L)], sem_rows))
                copies.append(pltpu.async_copy(
                    iembd_hbm.at[:, :, pl.ds(vi16, L)],
                    iseg_v.at[:, :, pl.ds(dj, L)], sem_rows))
                if (t + 1) % KB == 0:
                    for c in copies:
                        c.wait()
                    copies = []
            return carry2

        lax.fori_loop(0, CH // L, gather_batch, 0)

        def group(gi, carry2):
            j0 = c0 + gi * L
            uvec = uidx_v[pl.ds(j0, L)]
            ivec = iidx_v[pl.ds(j0, L)]
            slot = (gi * L + lanes) * L
            ucol = slot + (uvec & (L - 1))
            icol = slot + (ivec & (L - 1))
            uboff = (j0 + lanes) * L + (uvec & (L - 1))
            iboff = (j0 + lanes) * L + (ivec & (L - 1))
            acc = (plsc.load_gather(ubias_v, [uboff])
                   + plsc.load_gather(ibias_v, [iboff]) + oab)
            for a in range(PLANES):
                af = jnp.full((L,), a, jnp.int32)
                for r in range(SUB):
                    rf = jnp.full((L,), r, jnp.int32)
                    acc = acc + (plsc.load_gather(useg_v, [af, rf, ucol])
                                 * plsc.load_gather(iseg_v, [af, rf, icol]))
            out_v[pl.ds(j0, L)] = acc
            return carry2

        lax.fori_loop(0, CH // L, group, 0)
        return carry

    lax.fori_loop(0, NCH, chunk, 0)

    pltpu.sync_copy(out_v, out_hbm.at[pl.ds(base, PW)])


def kernel(userIdx, itemIdx, uEmbd, iEmbd, uBias, iBias, overAllBias):
    uidx = userIdx.astype(jnp.int32)
    iidx = itemIdx.astype(jnp.int32)
    ut = uEmbd.T.reshape(PLANES, SUB, uEmbd.shape[0])
    it = iEmbd.T.reshape(PLANES, SUB, iEmbd.shape[0])
    ubias = uBias.reshape(-1)
    ibias = iBias.reshape(-1)
    oab = jnp.broadcast_to(overAllBias.astype(jnp.float32), (L,))
    return _sc_predict(uidx, iidx, ut, it, ubias, ibias, oab)


# speedup vs baseline: 3.5973x; 1.3103x over previous
import functools

import jax
import jax.numpy as jnp
from jax import lax
from jax.experimental import pallas as pl
from jax.experimental.pallas import tpu as pltpu
from jax.experimental.pallas import tpu_sc as plsc

NC = 2
NS = 16
L = 16
NW = NC * NS

B = 16384
D = 32
SUB = 8
PLANES = D // SUB
PW = B // NW
GROUPS = PW // L
EPB = SUB
NBLK = PW // EPB

_mesh = plsc.VectorSubcoreMesh(core_axis_name="c", subcore_axis_name="s")


@functools.partial(
    pl.kernel,
    out_type=jax.ShapeDtypeStruct((B,), jnp.float32),
    mesh=_mesh,
    scratch_types=[
        pltpu.VMEM((PW,), jnp.int32),
        pltpu.VMEM((PW,), jnp.int32),
        pltpu.VMEM((2, PLANES, SUB, L * EPB), jnp.float32),
        pltpu.VMEM((2, PLANES, SUB, L * EPB), jnp.float32),
        pltpu.VMEM((PW * L,), jnp.float32),
        pltpu.VMEM((PW * L,), jnp.float32),
        pltpu.VMEM((L,), jnp.float32),
        pltpu.VMEM((PW,), jnp.float32),
        pltpu.SemaphoreType.DMA,
        pltpu.SemaphoreType.DMA,
    ],
    compiler_params=pltpu.CompilerParams(needs_layout_passes=False),
)
def _sc_predict(uidx_hbm, iidx_hbm, uembd_hbm, iembd_hbm, ubias_hbm,
                ibias_hbm, oab_hbm, out_hbm,
                uidx_v, iidx_v, useg_v, iseg_v, ubias_v, ibias_v,
                oab_v, out_v, sem_rows, sem_bias):
    wid = lax.axis_index("s") * NC + lax.axis_index("c")
    base = wid * PW

    pltpu.sync_copy(uidx_hbm.at[pl.ds(base, PW)], uidx_v)
    pltpu.sync_copy(iidx_hbm.at[pl.ds(base, PW)], iidx_v)
    pltpu.sync_copy(oab_hbm, oab_v)

    oab = oab_v[...]
    lanes = lax.iota(jnp.int32, L)

    def group(gi, carry):
        j0 = gi * L
        uvec = uidx_v[pl.ds(j0, L)]
        ivec = iidx_v[pl.ds(j0, L)]
        copies = []
        for t in range(L):
            j = j0 + t
            blk_t = t // EPB
            e = t % EPB
            vu16 = pl.multiple_of((uvec[t] >> 4) << 4, L)
            vi16 = pl.multiple_of((ivec[t] >> 4) << 4, L)
            de = pl.multiple_of(e * L, L)
            dj = pl.multiple_of(j * L, L)
            copies.append(pltpu.async_copy(
                uembd_hbm.at[:, :, pl.ds(vu16, L)],
                useg_v.at[blk_t, :, :, pl.ds(de, L)], sem_rows))
            copies.append(pltpu.async_copy(
                iembd_hbm.at[:, :, pl.ds(vi16, L)],
                iseg_v.at[blk_t, :, :, pl.ds(de, L)], sem_rows))
            copies.append(pltpu.async_copy(
                ubias_hbm.at[pl.ds(vu16, L)],
                ubias_v.at[pl.ds(dj, L)], sem_bias))
            copies.append(pltpu.async_copy(
                ibias_hbm.at[pl.ds(vi16, L)],
                ibias_v.at[pl.ds(dj, L)], sem_bias))
        for c in copies:
            c.wait()

        blk = lanes >> 3
        minor_u = ((lanes & (EPB - 1)) << 4) + (uvec & (L - 1))
        minor_i = ((lanes & (EPB - 1)) << 4) + (ivec & (L - 1))
        uboff = (j0 + lanes) * L + (uvec & (L - 1))
        iboff = (j0 + lanes) * L + (ivec & (L - 1))
        acc = (plsc.load_gather(ubias_v, [uboff])
               + plsc.load_gather(ibias_v, [iboff]) + oab)
        for a in range(PLANES):
            af = jnp.full((L,), a, jnp.int32)
            for r in range(SUB):
                rf = jnp.full((L,), r, jnp.int32)
                acc = acc + (plsc.load_gather(useg_v, [blk, af, rf, minor_u])
                             * plsc.load_gather(iseg_v, [blk, af, rf, minor_i]))
        out_v[pl.ds(j0, L)] = acc
        return carry

    lax.fori_loop(0, GROUPS, group, 0)

    pltpu.sync_copy(out_v, out_hbm.at[pl.ds(base, PW)])


def kernel(userIdx, itemIdx, uEmbd, iEmbd, uBias, iBias, overAllBias):
    uidx = userIdx.astype(jnp.int32)
    iidx = itemIdx.astype(jnp.int32)
    ut = uEmbd.T.reshape(PLANES, SUB, uEmbd.shape[0])
    it = iEmbd.T.reshape(PLANES, SUB, iEmbd.shape[0])
    ubias = uBias.reshape(-1)
    ibias = iBias.reshape(-1)
    oab = jnp.broadcast_to(overAllBias.astype(jnp.float32), (L,))
    return _sc_predict(uidx, iidx, ut, it, ubias, ibias, oab)
